# Initial kernel scaffold; baseline (speedup 1.0000x reference)
import jax
import jax.numpy as jnp
from jax.experimental import pallas as pl


def kernel(x, Wg, W1, b1, W2, b2):
    raise NotImplementedError("write your pallas kernel here")


# speedup vs baseline: 2.0162x; 2.0162x over previous
import functools
import math

import jax
import jax.numpy as jnp
from jax.experimental import pallas as pl
from jax.experimental.pallas import tpu as pltpu

T = 2048
D = 1024
F = 4096
E = 8
K = 2
CAP = int(math.ceil(T * K / E * 1.2))
CP = 640

_NEG = -3.0e38


def _router_body(x_ref, wg_ref, out_ref, oh_ref, excl_ref):
    x = x_ref[...]
    logits = jax.lax.dot_general(x, wg_ref[...], (((1,), (0,)), ((), ())))
    iota = jax.lax.broadcasted_iota(jnp.int32, (T, E), 1)
    m0 = jnp.max(logits, axis=1, keepdims=True)
    a0 = jnp.min(jnp.where(logits == m0, iota, E), axis=1, keepdims=True)
    l1 = jnp.where(iota == a0, _NEG, logits)
    m1 = jnp.max(l1, axis=1, keepdims=True)
    a1 = jnp.min(jnp.where(l1 == m1, iota, E), axis=1, keepdims=True)
    ed = jnp.exp(m1 - m0)
    g0 = 1.0 / (1.0 + ed)
    g1 = 1.0 - g0
    oh_ref[...] = ((iota == a0) | (iota == a1)).astype(jnp.float32)

    chunk = 128
    r = jax.lax.broadcasted_iota(jnp.int32, (chunk, chunk), 0)
    c = jax.lax.broadcasted_iota(jnp.int32, (chunk, chunk), 1)
    ltri = (r > c).astype(jnp.float32)

    def body(i, carry):
        blk = oh_ref[pl.ds(i * chunk, chunk), :]
        excl_ref[pl.ds(i * chunk, chunk), :] = (
            jax.lax.dot_general(ltri, blk, (((1,), (0,)), ((), ()))) + carry
        )
        return carry + jnp.sum(blk, axis=0, keepdims=True)

    jax.lax.fori_loop(0, T // chunk, body, jnp.zeros((1, E), jnp.float32))

    excl = excl_ref[...]
    p0 = jnp.sum(jnp.where(iota == a0, excl, 0.0), axis=1, keepdims=True)
    p1 = jnp.sum(jnp.where(iota == a1, excl, 0.0), axis=1, keepdims=True)
    k0 = (p0 < CAP).astype(jnp.float32)
    k1 = (p1 < CAP).astype(jnp.float32)
    out = jnp.concatenate(
        [
            a0.astype(jnp.float32),
            a1.astype(jnp.float32),
            p0,
            p1,
            g0 * k0,
            g1 * k1,
            k0,
            k1,
        ],
        axis=1,
    )
    out_ref[...] = out


def _dispatch_body(ft_ref, x_ref, buf_ref):
    e = pl.program_id(0).astype(jnp.float32)
    e0 = ft_ref[0:1, :]
    e1 = ft_ref[1:2, :]
    p0 = ft_ref[2:3, :]
    p1 = ft_ref[3:4, :]
    k0 = ft_ref[6:7, :]
    k1 = ft_ref[7:8, :]
    ci = jax.lax.broadcasted_iota(jnp.int32, (CP, T), 0).astype(jnp.float32)
    m0 = ((e0 == e) & (p0 == ci) & (k0 > 0.0)).astype(jnp.float32)
    m1 = ((e1 == e) & (p1 == ci) & (k1 > 0.0)).astype(jnp.float32)
    buf_ref[0] = jax.lax.dot_general(
        m0 + m1, x_ref[...], (((1,), (0,)), ((), ()))
    )


def _ffn_body(buf_ref, w1_ref, b1_ref, w2_ref, b2_ref, out_ref):
    f = pl.program_id(1)

    @pl.when(f == 0)
    def _init():
        out_ref[0] = jnp.broadcast_to(b2_ref[0], (CP, D))

    h = jax.lax.dot_general(buf_ref[0], w1_ref[0], (((1,), (0,)), ((), ())))
    h = h + b1_ref[0]
    h3 = h * h * h
    g = 0.5 * h * (1.0 + jnp.tanh(0.7978845608028654 * (h + 0.044715 * h3)))
    out_ref[0] += jax.lax.dot_general(g, w2_ref[0], (((1,), (0,)), ((), ())))


def _combine_body(f_ref, y_ref, out_ref):
    e = pl.program_id(0).astype(jnp.float32)

    @pl.when(pl.program_id(0) == 0)
    def _init():
        out_ref[...] = jnp.zeros((T, D), jnp.float32)

    e0 = f_ref[:, 0:1]
    e1 = f_ref[:, 1:2]
    p0 = f_ref[:, 2:3]
    p1 = f_ref[:, 3:4]
    w0 = f_ref[:, 4:5]
    w1 = f_ref[:, 5:6]
    ci = jax.lax.broadcasted_iota(jnp.int32, (T, CP), 1).astype(jnp.float32)
    g = w0 * ((e0 == e) & (p0 == ci)).astype(jnp.float32)
    g = g + w1 * ((e1 == e) & (p1 == ci)).astype(jnp.float32)
    out_ref[...] += jax.lax.dot_general(
        g, y_ref[0], (((1,), (0,)), ((), ()))
    )


@jax.jit
def kernel(x, Wg, W1, b1, W2, b2):
    fields = pl.pallas_call(
        _router_body,
        out_shape=jax.ShapeDtypeStruct((T, E), jnp.float32),
        scratch_shapes=[
            pltpu.VMEM((T, E), jnp.float32),
            pltpu.VMEM((T, E), jnp.float32),
        ],
    )(x, Wg)
    ft = fields.T

    buf = pl.pallas_call(
        _dispatch_body,
        grid=(E,),
        in_specs=[
            pl.BlockSpec((E, T), lambda e: (0, 0)),
            pl.BlockSpec((T, D), lambda e: (0, 0)),
        ],
        out_specs=pl.BlockSpec((1, CP, D), lambda e: (e, 0, 0)),
        out_shape=jax.ShapeDtypeStruct((E, CP, D), jnp.float32),
    )(ft, x)

    fb = 512
    yexp = pl.pallas_call(
        _ffn_body,
        grid=(E, F // fb),
        in_specs=[
            pl.BlockSpec((1, CP, D), lambda e, f: (e, 0, 0)),
            pl.BlockSpec((1, D, fb), lambda e, f: (e, 0, f)),
            pl.BlockSpec((1, 1, fb), lambda e, f: (e, 0, f)),
            pl.BlockSpec((1, fb, D), lambda e, f: (e, f, 0)),
            pl.BlockSpec((1, 1, D), lambda e, f: (e, 0, 0)),
        ],
        out_specs=pl.BlockSpec((1, CP, D), lambda e, f: (e, 0, 0)),
        out_shape=jax.ShapeDtypeStruct((E, CP, D), jnp.float32),
    )(buf, W1, b1.reshape(E, 1, F), W2, b2.reshape(E, 1, D))

    y = pl.pallas_call(
        _combine_body,
        grid=(E,),
        in_specs=[
            pl.BlockSpec((T, E), lambda e: (0, 0)),
            pl.BlockSpec((1, CP, D), lambda e: (e, 0, 0)),
        ],
        out_specs=pl.BlockSpec((T, D), lambda e: (0, 0)),
        out_shape=jax.ShapeDtypeStruct((T, D), jnp.float32),
    )(fields, yexp)
    return y
